# baseline (speedup 1.0000x reference)
import functools

import jax
import jax.numpy as jnp
from jax import lax
from jax.experimental import pallas as pl
from jax.experimental.pallas import tpu as pltpu
from jax.experimental.pallas import tpu_sc as plsc

N_NODES = 10000
D = 128
N_EDGES = 320000

NC = 2
NS = 16
NW = NC * NS
LANES = 16

CHUNK = 128
CPW = 80
N_CHUNKS = CPW * NW
E_PAD = N_CHUNKS * CHUNK
ROWS_PER_TILE = 624
ZCHUNK = 208
EXTRA_BASE = ROWS_PER_TILE * NS


def _sc_body(src_h, dst_h, val_h, x_h, out_h,
             acc, srcv, dstb, valb, rows0, rows1, sem0, sem1):
    c = lax.axis_index("c")
    s = lax.axis_index("s")
    wid = c * NS + s
    start = wid * CPW

    pltpu.sync_copy(src_h.at[pl.ds(start, CPW)], srcv)

    @pl.loop(0, CHUNK)
    def _(i):
        for cv in range(D // LANES):
            rows0[i, pl.ds(cv * LANES, LANES)] = jnp.zeros((LANES,), jnp.float32)

    rb = s * ROWS_PER_TILE
    for k in range(ROWS_PER_TILE // CHUNK):
        pltpu.sync_copy(rows0, acc.at[pl.ds(rb + k * CHUNK, CHUNK)])
    rem = ROWS_PER_TILE % CHUNK
    pltpu.sync_copy(rows0.at[pl.ds(0, rem)],
                    acc.at[pl.ds(rb + ROWS_PER_TILE - rem, rem)])

    @pl.when(s == NS - 1)
    def _():
        pltpu.sync_copy(rows0.at[pl.ds(0, N_NODES - EXTRA_BASE)],
                        acc.at[pl.ds(EXTRA_BASE, N_NODES - EXTRA_BASE)])

    plsc.subcore_barrier()

    rowsb = (rows0, rows1)
    semb = (sem0, sem1)

    def fire(j, b):
        sem = semb[b]
        pltpu.async_copy(x_h.at[srcv.at[j]], rowsb[b], sem)
        ebase = (start + j) * CHUNK
        pltpu.async_copy(dst_h.at[pl.ds(ebase, CHUNK)], dstb.at[b], sem)
        pltpu.async_copy(val_h.at[pl.ds(ebase, CHUNK)], valb.at[b], sem)

    def process(j, b):
        rows, sem = rowsb[b], semb[b]
        ebase = (start + j) * CHUNK
        pltpu.make_async_copy(x_h.at[srcv.at[j]], rows, sem).wait()
        pltpu.make_async_copy(dst_h.at[pl.ds(ebase, CHUNK)], dstb.at[b], sem).wait()
        pltpu.make_async_copy(val_h.at[pl.ds(ebase, CHUNK)], valb.at[b], sem).wait()

        @pl.loop(0, CHUNK // LANES)
        def _(g):
            vv = valb[b, pl.ds(g * LANES, LANES)]
            for l in range(LANES):
                v = vv[l]
                e = g * LANES + l
                for cv in range(D // LANES):
                    sl = pl.ds(cv * LANES, LANES)
                    rows[e, sl] = rows[e, sl] * v

        pltpu.sync_copy(rows, acc.at[dstb.at[b]], add=True)

    fire(0, 0)

    @pl.loop(0, CPW // 2)
    def _(j2):
        j = 2 * j2
        fire(j + 1, 1)
        process(j, 0)

        @pl.when(j + 2 < CPW)
        def _():
            fire(j + 2, 0)

        process(j + 1, 1)

    plsc.subcore_barrier()
    rb = s * ROWS_PER_TILE
    pltpu.sync_copy(acc.at[pl.ds(rb, ROWS_PER_TILE)],
                    out_h.at[c, pl.ds(rb, ROWS_PER_TILE)])

    @pl.when(s == NS - 1)
    def _():
        pltpu.sync_copy(acc.at[pl.ds(EXTRA_BASE, N_NODES - EXTRA_BASE)],
                        out_h.at[c, pl.ds(EXTRA_BASE, N_NODES - EXTRA_BASE)])


_sc_scatter = pl.kernel(
    _sc_body,
    out_type=jax.ShapeDtypeStruct((NC, N_NODES, D), jnp.float32),
    mesh=plsc.VectorSubcoreMesh(
        core_axis_name="c", subcore_axis_name="s",
        num_cores=NC, num_subcores=NS),
    scratch_types=[
        pltpu.VMEM_SHARED((N_NODES, D), jnp.float32),
        pltpu.VMEM((CPW, CHUNK), jnp.int32),
        pltpu.VMEM((2, CHUNK), jnp.int32),
        pltpu.VMEM((2, CHUNK), jnp.float32),
        pltpu.VMEM((CHUNK, D), jnp.float32),
        pltpu.VMEM((CHUNK, D), jnp.float32),
        pltpu.SemaphoreType.DMA,
        pltpu.SemaphoreType.DMA,
    ],
)


BR = 400


def _mm_body(p_ref, w_ref, b_ref, o_ref):
    z = p_ref[0] + p_ref[1]
    o_ref[...] = (
        jnp.dot(z, w_ref[...], preferred_element_type=jnp.float32) + b_ref[...]
    )


_tc_matmul = pl.pallas_call(
    _mm_body,
    grid=(N_NODES // BR,),
    in_specs=[
        pl.BlockSpec((NC, BR, D), lambda i: (0, i, 0)),
        pl.BlockSpec((D, D), lambda i: (0, 0)),
        pl.BlockSpec((1, D), lambda i: (0, 0)),
    ],
    out_specs=pl.BlockSpec((BR, D), lambda i: (i, 0)),
    out_shape=jax.ShapeDtypeStruct((N_NODES, D), jnp.float32),
)


@jax.jit
def kernel(adjacency_indices, adjacency_values, input_features, W, bias):
    dst = adjacency_indices[0]
    src = adjacency_indices[1]
    pad = E_PAD - N_EDGES
    src_p = jnp.concatenate([src, jnp.zeros((pad,), jnp.int32)]).reshape(
        N_CHUNKS, CHUNK)
    dst_p = jnp.concatenate([dst, jnp.zeros((pad,), jnp.int32)])
    val_p = jnp.concatenate([adjacency_values, jnp.zeros((pad,), jnp.float32)])
    partials = _sc_scatter(src_p, dst_p, val_p, input_features)
    return _tc_matmul(partials, W, bias.reshape(1, D))
